# baseline (speedup 1.0000x reference)
import jax
import jax.numpy as jnp
import numpy as np
from jax.experimental import pallas as pl

_B = 2048
_BBLK = 128
_GRID = _B // _BBLK
_K = 64
_NPOS = 4
_CD = 4

_PERM16 = np.array([(i % 4) * 4 + i // 4 for i in range(16)])
_PERM1024 = np.array([(i % 64) * 16 + i // 64 for i in range(1024)])


def _sel_conv(h, oh, k, p, s):
    ih = np.arange(h)[:, None, None]
    kh = np.arange(k)[None, :, None]
    ohh = np.arange(oh)[None, None, :]
    return (ih == s * ohh + kh - p).astype(np.float32)


def _sel_convt(h, oh, k, p, s):
    ih = np.arange(h)[:, None, None]
    kh = np.arange(k)[None, :, None]
    ohh = np.arange(oh)[None, None, :]
    return (ohh == s * ih + kh - p).astype(np.float32)


def _mat_conv(w, h, oh, p=2, s=2):
    sel = jnp.asarray(_sel_conv(h, oh, w.shape[2], p, s))
    m = jnp.einsum('ocij,hia,wjb->chwoab', w, sel, sel)
    return m.reshape(w.shape[1] * h * h, w.shape[0] * oh * oh)


def _mat_convt(w, h, oh, p, s=2):
    sel = jnp.asarray(_sel_convt(h, oh, w.shape[2], p, s))
    m = jnp.einsum('coij,hia,wjb->chwoab', w, sel, sel)
    return m.reshape(w.shape[0] * h * h, w.shape[1] * oh * oh)


def _vq_body(xr, m1, b1, m2, b2, m3, b3, w4c, b4r,
             mt1, t1, mt2, t2, mt3, t3, mt4, t4, dflat, dcb,
             lbuf, idx):
    step = pl.program_id(0)
    x = xr[...]

    def lin(a, m, b):
        return jnp.dot(a, m[...], preferred_element_type=jnp.float32) + b[0:1, :]

    y = jnp.maximum(lin(x.astype(jnp.bfloat16), m1, b1), 0.0).astype(jnp.bfloat16)
    y = jnp.maximum(lin(y, m2, b2), 0.0).astype(jnp.bfloat16)
    y = jnp.maximum(lin(y, m3, b3), 0.0).astype(jnp.bfloat16)

    zero64 = jnp.zeros((_BBLK, 64), jnp.bfloat16)
    zpos = []
    for oh in range(2):
        for ow in range(2):
            pieces = []
            for kh in range(5):
                ih = 2 * oh + kh - 2
                for kw in range(5):
                    iw = 2 * ow + kw - 2
                    if 0 <= ih < 4 and 0 <= iw < 4:
                        s = (ih * 4 + iw) * 64
                        pieces.append(y[:, s:s + 64])
                    else:
                        pieces.append(zero64)
            col = jnp.concatenate(pieces, axis=1)
            zpos.append(
                jnp.dot(col, w4c[...], preferred_element_type=jnp.float32)
                + b4r[0:1, :])
    z = jnp.concatenate(zpos, axis=1)

    d0 = dflat[0:1, :]
    idxs, vals = [], []
    for pos in range(_NPOS):
        zp = z[:, _CD * pos:_CD * pos + _CD]
        zrep = jnp.concatenate(
            [jnp.broadcast_to(zp[:, c:c + 1], (_BBLK, _K)) for c in range(_CD)],
            axis=1)
        df = zrep - d0
        df2 = df * df
        dist2 = (df2[:, 0:64] + df2[:, 64:128]
                 + df2[:, 128:192] + df2[:, 192:256])
        dist = jnp.sqrt(dist2)
        dmin = jnp.min(dist, axis=1, keepdims=True)
        ji = jax.lax.broadcasted_iota(jnp.int32, (_BBLK, _K), 1)
        ix = jnp.min(jnp.where(dist == dmin, ji, _K), axis=1, keepdims=True)
        onehot = (ji == ix).astype(jnp.float32)
        vals.append(jnp.dot(onehot, dcb[...], preferred_element_type=jnp.float32))
        idxs.append(ix)
    val = jnp.concatenate(vals, axis=1)
    idx[...] = jnp.concatenate(idxs, axis=1)

    dv = val - z
    dsum = jnp.sum(jnp.sum(dv * dv, axis=1, keepdims=True), axis=0, keepdims=True)

    y = jnp.maximum(lin(val.astype(jnp.bfloat16), mt1, t1), 0.0).astype(jnp.bfloat16)
    y = jnp.maximum(lin(y, mt2, t2), 0.0).astype(jnp.bfloat16)
    y = jnp.maximum(lin(y, mt3, t3), 0.0).astype(jnp.bfloat16)
    f = lin(y, mt4, t4)
    r = f - x
    rsum = jnp.sum(jnp.sum(r * r, axis=1, keepdims=True), axis=0, keepdims=True)

    @pl.when(step == 0)
    def _init():
        lbuf[...] = jnp.zeros_like(lbuf)

    lbuf[...] += jnp.concatenate(
        [jnp.broadcast_to(rsum, (1, 128)),
         jnp.broadcast_to(dsum, (1, 128)),
         jnp.zeros((6, 128), jnp.float32)], axis=0)


def kernel(x, w1, b1, w2, b2, w3, b3, w4, b4,
           tw1, tb1, tw2, tb2, tw3, tb3, tw4, tb4, dict_w):
    xr = x.reshape(_B, 784)
    m1 = _mat_conv(w1, 28, 14).astype(jnp.bfloat16)
    m2 = _mat_conv(w2, 14, 7).astype(jnp.bfloat16)
    m3 = _mat_conv(w3, 7, 4)[:, _PERM1024].astype(jnp.bfloat16)
    w4c = jnp.transpose(w4, (2, 3, 1, 0)).reshape(1600, 4).astype(jnp.bfloat16)
    mt1 = _mat_convt(tw1, 2, 3, 2)[_PERM16, :].astype(jnp.bfloat16)
    mt2 = _mat_convt(tw2, 3, 5, 2).astype(jnp.bfloat16)
    mt3 = _mat_convt(tw3, 5, 13, 0).astype(jnp.bfloat16)
    mt4 = _mat_convt(tw4, 13, 28, 0).astype(jnp.bfloat16)

    def bb(v):
        return jnp.broadcast_to(v[None, :].astype(jnp.float32), (8, v.shape[0]))

    b1v = bb(jnp.repeat(b1, 196))
    b2v = bb(jnp.repeat(b2, 49))
    b3v = bb(jnp.tile(b3, 16))
    b4r = bb(b4)
    t1v = bb(jnp.repeat(tb1, 9))
    t2v = bb(jnp.repeat(tb2, 25))
    t3v = bb(jnp.repeat(tb3, 169))
    t4v = bb(jnp.repeat(tb4, 784))
    dflat = bb(dict_w.T.reshape(256))
    dcb = dict_w.astype(jnp.float32)

    def full(a):
        return pl.BlockSpec(a.shape, lambda i: (0,) * a.ndim)

    ins = (xr, m1, b1v, m2, b2v, m3, b3v, w4c, b4r,
           mt1, t1v, mt2, t2v, mt3, t3v, mt4, t4v, dflat, dcb)
    in_specs = [pl.BlockSpec((_BBLK, 784), lambda i: (i, 0))] + [full(a) for a in ins[1:]]
    lbuf, idxs = pl.pallas_call(
        _vq_body,
        grid=(_GRID,),
        in_specs=in_specs,
        out_specs=[pl.BlockSpec((8, 128), lambda i: (0, 0)),
                   pl.BlockSpec((_BBLK, _NPOS), lambda i: (i, 0))],
        out_shape=[jax.ShapeDtypeStruct((8, 128), jnp.float32),
                   jax.ShapeDtypeStruct((_B, _NPOS), jnp.int32)],
    )(*ins)
    loss_rec = lbuf[0, 0] / _B
    dict_loss = lbuf[1, 0] / 8192.0
    enc_loss = lbuf[1, 0] / 32768.0
    var_loss = jnp.zeros((1,), jnp.float32)
    index = idxs.reshape(_B, 2, 2)
    return (loss_rec, dict_loss, enc_loss, var_loss, index)
